# baseline (speedup 1.0000x reference)
import functools

import jax
import jax.numpy as jnp
from jax import lax
from jax.experimental import pallas as pl
from jax.experimental.pallas import tpu as pltpu
from jax.experimental.pallas import tpu_sc as plsc

_N = 10000
_D = 128
_NP = 10240
_E = 320000
_C = 128
_NTILES = 32
_NC = 80
_NH = 40
_EPT = _NC * _C
_EP = _NTILES * _EPT
_RPT = _NP // 16


def _sc_agg_body(with_counts, *refs):
    if with_counts:
        (x_hbm, src_hbm, dst_hbm, out_hbm, cnt_hbm,
         acc_sh, cnt_sh, sidx_v, didx_v, rows0, rows1,
         ones_v, zs_v, sem0, sem1) = refs
    else:
        (x_hbm, src_hbm, dst_hbm, out_hbm,
         acc_sh, sidx_v, didx_v, rows0, rows1, sem0, sem1) = refs

    cid = lax.axis_index("c")
    sid = lax.axis_index("s")
    w = cid * 16 + sid

    zero16 = jnp.zeros((16,), jnp.float32)

    def _zrow(i, c):
        for j in range(8):
            rows0[i, pl.ds(j * 16, 16)] = zero16
        return c
    lax.fori_loop(0, _C, _zrow, 0)

    base = sid * _RPT
    for r in range(_RPT // _C):
        pltpu.sync_copy(rows0, acc_sh.at[pl.ds(base + r * _C, _C)])

    if with_counts:
        ones16 = jnp.full((16,), 1.0, jnp.float32)
        def _zcnt(i, c):
            zs_v[pl.ds(i * 16, 16)] = zero16
            return c
        lax.fori_loop(0, _RPT // 16, _zcnt, 0)
        for j in range(_C // 16):
            ones_v[pl.ds(j * 16, 16)] = ones16
        pltpu.sync_copy(zs_v, cnt_sh.at[pl.ds(base, _RPT)])

    plsc.subcore_barrier()

    def _half(g, rows, sem):
        pltpu.make_async_copy(x_hbm.at[sidx_v.at[g]], rows, sem).wait()
        pltpu.sync_copy(rows, acc_sh.at[didx_v.at[g]], add=True)
        if with_counts:
            pltpu.sync_copy(ones_v, cnt_sh.at[didx_v.at[g]], add=True)

        @pl.when(g + 2 < _NH)
        def _():
            pltpu.make_async_copy(x_hbm.at[sidx_v.at[g + 2]], rows, sem
                                  ).start()

    def _pair(gg, c):
        _half(2 * gg, rows0, sem0)
        _half(2 * gg + 1, rows1, sem1)
        return c

    for s in range(_NC // _NH):
        off = w * _NC + s * _NH
        pltpu.sync_copy(src_hbm.at[pl.ds(off, _NH)], sidx_v)
        pltpu.sync_copy(dst_hbm.at[pl.ds(off, _NH)], didx_v)
        pltpu.make_async_copy(x_hbm.at[sidx_v.at[0]], rows0, sem0).start()
        pltpu.make_async_copy(x_hbm.at[sidx_v.at[1]], rows1, sem1).start()
        lax.fori_loop(0, _NH // 2, _pair, 0)

    plsc.subcore_barrier()

    out_base = cid * _NP + base
    for r in range(_RPT // _C):
        pltpu.sync_copy(acc_sh.at[pl.ds(base + r * _C, _C)],
                        out_hbm.at[pl.ds(out_base + r * _C, _C)])
    if with_counts:
        pltpu.sync_copy(cnt_sh.at[pl.ds(base, _RPT)],
                        cnt_hbm.at[pl.ds(out_base, _RPT)])


def _make_sc_agg(with_counts):
    mesh = plsc.VectorSubcoreMesh(core_axis_name="c", subcore_axis_name="s")
    out_type = [jax.ShapeDtypeStruct((2 * _NP, _D), jnp.float32)]
    scratch = [
        pltpu.VMEM_SHARED((_NP, _D), jnp.float32),
        pltpu.VMEM((_NH, _C), jnp.int32),
        pltpu.VMEM((_NH, _C), jnp.int32),
        pltpu.VMEM((_C, _D), jnp.float32),
        pltpu.VMEM((_C, _D), jnp.float32),
        pltpu.SemaphoreType.DMA,
        pltpu.SemaphoreType.DMA,
    ]
    if with_counts:
        out_type.append(jax.ShapeDtypeStruct((2 * _NP,), jnp.float32))
        scratch.insert(1, pltpu.VMEM_SHARED((_NP,), jnp.float32))
        scratch.insert(6, pltpu.VMEM((_C,), jnp.float32))
        scratch.insert(7, pltpu.VMEM((_RPT,), jnp.float32))
    return pl.kernel(
        functools.partial(_sc_agg_body, with_counts),
        mesh=mesh,
        out_type=out_type,
        scratch_types=scratch,
    )


_sc_agg_counts = _make_sc_agg(True)
_sc_agg_plain = _make_sc_agg(False)


def _dense_body(apply_relu, p0, p1, c0, c1, xr, wl, wr, b, out):
    cnt = c0[...] + c1[...]
    inv = 1.0 / jnp.maximum(cnt, 1.0)
    agg = (p0[...] + p1[...]) * inv
    acc = (jnp.dot(agg, wl[...], preferred_element_type=jnp.float32)
           + jnp.dot(xr[...], wr[...], preferred_element_type=jnp.float32)
           + b[...])
    if apply_relu:
        acc = jnp.maximum(acc, 0.0)
    out[...] = acc


_BLK = 1024


def _make_dense(apply_relu):
    row = pl.BlockSpec((_BLK, _D), lambda i: (i, 0))
    col = pl.BlockSpec((_BLK, 1), lambda i: (i, 0))
    full = pl.BlockSpec((_D, _D), lambda i: (0, 0))
    bias = pl.BlockSpec((1, _D), lambda i: (0, 0))
    return pl.pallas_call(
        functools.partial(_dense_body, apply_relu),
        grid=(_NP // _BLK,),
        in_specs=[row, row, col, col, row, full, full, bias],
        out_specs=row,
        out_shape=jax.ShapeDtypeStruct((_NP, _D), jnp.float32),
    )


_dense_relu = _make_dense(True)
_dense_lin = _make_dense(False)


def kernel(x, edge_index, W1_l, W1_r, b1, W2_l, W2_r, b2):
    pad_e = _EP - _E
    src = jnp.concatenate([edge_index[0], jnp.zeros((pad_e,), jnp.int32)])
    dst = jnp.concatenate([edge_index[1],
                           jnp.full((pad_e,), _N, jnp.int32)])
    src = src.reshape(_NTILES * _NC, _C)
    dst = dst.reshape(_NTILES * _NC, _C)
    x_p = jnp.concatenate(
        [x, jnp.zeros((_NP - _N, _D), jnp.float32)], axis=0)
    b1_r = b1.reshape(1, _D)
    b2_r = b2.reshape(1, _D)

    sums1, cnts = _sc_agg_counts(x_p, src, dst)
    p0, p1 = sums1[:_NP], sums1[_NP:]
    c0 = cnts[:_NP].reshape(_NP, 1)
    c1 = cnts[_NP:].reshape(_NP, 1)

    h = _dense_relu(p0, p1, c0, c1, x_p, W1_l, W1_r, b1_r)

    (sums2,) = _sc_agg_plain(h, src, dst)
    q0, q1 = sums2[:_NP], sums2[_NP:]

    out = _dense_lin(q0, q1, c0, c1, h, W2_l, W2_r, b2_r)
    return out[:_N]


# speedup vs baseline: 1.3361x; 1.3361x over previous
import functools

import jax
import jax.numpy as jnp
from jax import lax
from jax.experimental import pallas as pl
from jax.experimental.pallas import tpu as pltpu
from jax.experimental.pallas import tpu_sc as plsc

_N = 10000
_D = 128
_H = 64
_NP = 10240
_E = 320000
_C = 128
_NC = 160
_EPT = _NC * _C
_EP = 16 * _EPT
_RPT = _NP // 16
_RING = 5


def _sc_agg_body(with_counts, *refs):
    refs = list(refs)
    x_hbm, src_hbm, dst_hbm, out_hbm = refs[:4]
    del refs[:4]
    if with_counts:
        cnt_hbm = refs.pop(0)
    acc_sh = refs.pop(0)
    if with_counts:
        cnt_sh = refs.pop(0)
    sidx_v, didx_v = refs.pop(0), refs.pop(0)
    rows = [refs.pop(0) for _ in range(_RING)]
    if with_counts:
        ones_v, zs_v = refs.pop(0), refs.pop(0)
    gsem = [refs.pop(0) for _ in range(_RING)]
    ssem = [refs.pop(0) for _ in range(_RING)]
    if with_counts:
        csem = refs.pop(0)
    r0 = rows[0]

    cid = lax.axis_index("c")
    sid = lax.axis_index("s")

    zero16 = jnp.zeros((16,), jnp.float32)

    def _zrow(i, c):
        for j in range(_H // 16):
            r0[i, pl.ds(j * 16, 16)] = zero16
        return c
    lax.fori_loop(0, _C, _zrow, 0)

    base = sid * _RPT
    for r in range(_RPT // _C):
        pltpu.sync_copy(r0, acc_sh.at[pl.ds(base + r * _C, _C)])

    if with_counts:
        ones16 = jnp.full((16,), 1.0, jnp.float32)
        def _zcnt(i, c):
            zs_v[pl.ds(i * 16, 16)] = zero16
            return c
        lax.fori_loop(0, _RPT // 16, _zcnt, 0)
        for j in range(_C // 16):
            ones_v[pl.ds(j * 16, 16)] = ones16

        pltpu.sync_copy(zs_v, cnt_sh.at[pl.ds(base, _RPT)])

    pltpu.sync_copy(src_hbm.at[pl.ds(cid * 16 * _NC + sid * _NC, _NC)],
                    sidx_v)
    pltpu.sync_copy(dst_hbm.at[pl.ds(sid * _NC, _NC)], didx_v)
    for k in range(_RING):
        pltpu.make_async_copy(x_hbm.at[sidx_v.at[k]], rows[k],
                              gsem[k]).start()

    plsc.subcore_barrier()

    def _scat(g, k):
        return pltpu.make_async_copy(rows[k], acc_sh.at[didx_v.at[g]],
                                     ssem[k])

    def _gath(g, k):
        return pltpu.make_async_copy(x_hbm.at[sidx_v.at[g]], rows[k],
                                     gsem[k])

    def _fire(qq):
        for k in range(_RING):
            g = _RING * qq + k
            _gath(g, k).wait()
            _scat(g, k).start(add=True)
            if with_counts:
                @pl.when(lax.rem(g, 2) == cid)
                def _():
                    pltpu.async_copy(ones_v, cnt_sh.at[didx_v.at[g]],
                                     csem, add=True)

    def _quad(qq, c):
        _fire(qq)
        for k in range(_RING):
            g = _RING * qq + k
            _scat(g, k).wait()
            _gath(g + _RING, k).start()
        return c

    lax.fori_loop(0, _NC // _RING - 1, _quad, 0)

    _fire(_NC // _RING - 1)
    for k in range(_RING):
        _scat(_NC - _RING + k, k).wait()
    if with_counts:
        def _drain(i, c):
            pltpu.make_async_copy(ones_v, cnt_sh.at[didx_v.at[0]],
                                  csem).wait()
            return c
        lax.fori_loop(0, _NC // 2, _drain, 0)

    plsc.subcore_barrier()

    out_base = cid * _NP + base
    for r in range(_RPT // _C):
        pltpu.sync_copy(acc_sh.at[pl.ds(base + r * _C, _C)],
                        out_hbm.at[pl.ds(out_base + r * _C, _C)])
    if with_counts:
        pltpu.sync_copy(cnt_sh.at[pl.ds(base, _RPT)],
                        cnt_hbm.at[pl.ds(out_base, _RPT)])


def _make_sc_agg(with_counts):
    mesh = plsc.VectorSubcoreMesh(core_axis_name="c", subcore_axis_name="s")
    out_type = [jax.ShapeDtypeStruct((2 * _NP, _H), jnp.float32)]
    scratch = [
        pltpu.VMEM_SHARED((_NP, _H), jnp.float32),
        pltpu.VMEM((_NC, _C), jnp.int32),
        pltpu.VMEM((_NC, _C), jnp.int32),
    ]
    scratch += [pltpu.VMEM((_C, _H), jnp.float32) for _ in range(_RING)]
    sems = [pltpu.SemaphoreType.DMA] * (2 * _RING)
    if with_counts:
        out_type.append(jax.ShapeDtypeStruct((2 * _NP,), jnp.float32))
        scratch.insert(1, pltpu.VMEM_SHARED((_NP,), jnp.float32))
        scratch.append(pltpu.VMEM((_C,), jnp.float32))
        scratch.append(pltpu.VMEM((_RPT,), jnp.float32))
        sems.append(pltpu.SemaphoreType.DMA)
    assert _NC % _RING == 0
    return pl.kernel(
        functools.partial(_sc_agg_body, with_counts),
        mesh=mesh,
        out_type=out_type,
        scratch_types=scratch + sems,
        compiler_params=pltpu.CompilerParams(use_tc_tiling_on_sc=False),
    )


_sc_agg_counts = _make_sc_agg(True)
_sc_agg_plain = _make_sc_agg(False)


_BLK = 1024


def _dense1_body(plo, phi, c0, c1, xr, wl, wr, b, out):
    j = pl.program_id(1)
    inv = 1.0 / jnp.maximum(c0[...] + c1[...], 1.0)
    agg = jnp.concatenate([plo[...], phi[...]], axis=1) * inv
    acc = (jnp.dot(agg, wl[...], preferred_element_type=jnp.float32)
           + jnp.dot(xr[...], wr[...], preferred_element_type=jnp.float32))
    bv = b[...]
    acc = acc + jnp.where(j == 0, bv[0], bv[1])[None, :]
    out[...] = jnp.maximum(acc, 0.0)


def _make_dense1():
    nrb = _NP // _BLK
    row = pl.BlockSpec((_BLK, _D), lambda i, j: (i, 0))
    lo = pl.BlockSpec((_BLK, _H), lambda i, j: (i, 0))
    hi = pl.BlockSpec((_BLK, _H), lambda i, j: (nrb + i, 0))
    clo = pl.BlockSpec((_BLK, 1), lambda i, j: (i, 0))
    chi = pl.BlockSpec((_BLK, 1), lambda i, j: (nrb + i, 0))
    wsp = pl.BlockSpec((_D, _H), lambda i, j: (j, 0))
    bsp = pl.BlockSpec((2, _H), lambda i, j: (0, 0))
    osp = pl.BlockSpec((_BLK, _H), lambda i, j: (j * nrb + i, 0))
    return pl.pallas_call(
        _dense1_body,
        grid=(nrb, 2),
        in_specs=[lo, hi, clo, chi, row, wsp, wsp, bsp],
        out_specs=osp,
        out_shape=jax.ShapeDtypeStruct((2 * _NP, _H), jnp.float32),
    )


def _dense2_body(plo, phi, c0, c1, xlo, xhi, wl, wr, b, out):
    inv = 1.0 / jnp.maximum(c0[...] + c1[...], 1.0)
    agg = jnp.concatenate([plo[...], phi[...]], axis=1) * inv
    xr = jnp.concatenate([xlo[...], xhi[...]], axis=1)
    acc = (jnp.dot(agg, wl[...], preferred_element_type=jnp.float32)
           + jnp.dot(xr[...], wr[...], preferred_element_type=jnp.float32)
           + b[...])
    out[...] = acc


def _make_dense2():
    nrb = _NP // _BLK
    lo = pl.BlockSpec((_BLK, _H), lambda i: (i, 0))
    hi = pl.BlockSpec((_BLK, _H), lambda i: (nrb + i, 0))
    clo = pl.BlockSpec((_BLK, 1), lambda i: (i, 0))
    chi = pl.BlockSpec((_BLK, 1), lambda i: (nrb + i, 0))
    full = pl.BlockSpec((_D, _D), lambda i: (0, 0))
    bsp = pl.BlockSpec((1, _D), lambda i: (0, 0))
    osp = pl.BlockSpec((_BLK, _D), lambda i: (i, 0))
    return pl.pallas_call(
        _dense2_body,
        grid=(nrb,),
        in_specs=[lo, hi, clo, chi, lo, hi, full, full, bsp],
        out_specs=osp,
        out_shape=jax.ShapeDtypeStruct((_NP, _D), jnp.float32),
    )


_dense1 = _make_dense1()
_dense2 = _make_dense2()


def kernel(x, edge_index, W1_l, W1_r, b1, W2_l, W2_r, b2):
    pad_e = _EP - _E
    src = jnp.concatenate([edge_index[0], jnp.zeros((pad_e,), jnp.int32)])
    dst = jnp.concatenate([edge_index[1],
                           jnp.full((pad_e,), _N, jnp.int32)])
    src2 = jnp.stack([src, src + _NP]).reshape(2 * 16 * _NC, _C)
    dst2 = dst.reshape(16 * _NC, _C)
    x_p = jnp.concatenate(
        [x, jnp.zeros((_NP - _N, _D), jnp.float32)], axis=0)
    xcat = jnp.concatenate([x_p[:, :_H], x_p[:, _H:]], axis=0)
    wl2 = jnp.concatenate([W1_l[:, :_H], W1_l[:, _H:]], axis=0)
    wr2 = jnp.concatenate([W1_r[:, :_H], W1_r[:, _H:]], axis=0)
    b1_r = b1.reshape(2, _H)
    b2_r = b2.reshape(1, _D)

    sums1, cnts = _sc_agg_counts(xcat, src2, dst2)
    cnt = cnts.reshape(2 * _NP, 1)

    h = _dense1(sums1, sums1, cnt, cnt, x_p, wl2, wr2, b1_r)

    (sums2,) = _sc_agg_plain(h, src2, dst2)

    out = _dense2(sums2, sums2, cnt, cnt, h, h, W2_l, W2_r, b2_r)
    return out[:_N]
